# baseline (speedup 1.0000x reference)
import functools

import jax
import jax.numpy as jnp
from jax import lax
from jax.experimental import pallas as pl
from jax.experimental.pallas import tpu as pltpu
from jax.experimental.pallas import tpu_sc as plsc

BS = 16
LEN_Q = 300
D_MODEL = 256
N_HEADS = 8
HEAD_DIM = 32
N_LEVELS = 4
P_TOTAL = 16
H = 40
W = 40
HW = H * W
N_ITEMS = BS * LEN_Q
ENTRIES = 4 * N_HEADS * P_TOTAL

NC, NS = 2, 16
NW = NC * NS
ITEMS_PER_W = N_ITEMS // NW
CHUNK = 3
N_CHUNKS = ITEMS_PER_W // CHUNK


def _prep_kernel(q_ref, rp_ref, wx_ref, wy_ref, wa_ref, bx_ref, by_ref,
                 ba_ref, idx_ref, wgt_ref):
    b = pl.program_id(0)
    q = q_ref[0]
    sox = jnp.dot(q, wx_ref[...], preferred_element_type=jnp.float32) + bx_ref[0]
    soy = jnp.dot(q, wy_ref[...], preferred_element_type=jnp.float32) + by_ref[0]
    logits = jnp.dot(q, wa_ref[...], preferred_element_type=jnp.float32) + ba_ref[0]

    aw_parts = []
    for h in range(N_HEADS):
        lg = logits[:, h * P_TOTAL:(h + 1) * P_TOTAL]
        e = jnp.exp(lg - jnp.max(lg, axis=1, keepdims=True))
        aw_parts.append(e / jnp.sum(e, axis=1, keepdims=True))
    aw = jnp.concatenate(aw_parts, axis=1)

    rp = rp_ref[0, :, 0, :]
    refx, refy = rp[:, 0:1], rp[:, 1:2]
    refw, refh = rp[:, 2:3], rp[:, 3:4]
    px = refx * W + sox * refw * (0.125 * W) - 0.5
    py = refy * H + soy * refh * (0.125 * H) - 0.5

    x0 = jnp.floor(px)
    y0 = jnp.floor(py)
    fx = px - x0
    fy = py - y0
    x1 = x0 + 1.0
    y1 = y0 + 1.0
    vx0 = ((x0 >= 0.0) & (x0 <= W - 1.0)).astype(jnp.float32)
    vx1 = ((x1 >= 0.0) & (x1 <= W - 1.0)).astype(jnp.float32)
    vy0 = ((y0 >= 0.0) & (y0 <= H - 1.0)).astype(jnp.float32)
    vy1 = ((y1 >= 0.0) & (y1 <= H - 1.0)).astype(jnp.float32)
    x0c = jnp.clip(x0, 0.0, W - 1.0).astype(jnp.int32)
    x1c = jnp.clip(x1, 0.0, W - 1.0).astype(jnp.int32)
    y0c = jnp.clip(y0, 0.0, H - 1.0).astype(jnp.int32)
    y1c = jnp.clip(y1, 0.0, H - 1.0).astype(jnp.int32)

    wx0 = (1.0 - fx) * vx0
    wx1 = fx * vx1
    wy0 = (1.0 - fy) * vy0
    wy1 = fy * vy1

    lane = lax.broadcasted_iota(jnp.int32, (LEN_Q, N_HEADS * P_TOTAL), 1)
    hh = lane >> 4
    lvl = (lane & 15) >> 2
    base = ((lvl * BS + b) * N_HEADS + hh) * HW

    row_y0 = base + y0c * W
    row_y1 = base + y1c * W
    idx_ref[0, :, 0, :] = row_y0 + x0c
    idx_ref[0, :, 1, :] = row_y0 + x1c
    idx_ref[0, :, 2, :] = row_y1 + x0c
    idx_ref[0, :, 3, :] = row_y1 + x1c
    wgt_ref[0, :, 0, :] = aw * wx0 * wy0
    wgt_ref[0, :, 1, :] = aw * wx1 * wy0
    wgt_ref[0, :, 2, :] = aw * wx0 * wy1
    wgt_ref[0, :, 3, :] = aw * wx1 * wy1


def _sc_gather_kernel(table_ref, idx_hbm, wgt_hbm, out_hbm,
                      idx_v, wgt_v, rows_v, out_v, rsem, isem, wsem):
    wid = lax.axis_index("s") * NC + lax.axis_index("c")
    w_base = wid * ITEMS_PER_W

    def issue_gathers(s, idxv_s):
        for i in range(CHUNK):
            for c in range(4):
                pltpu.async_copy(table_ref.at[idxv_s.at[i, c]],
                                 rows_v.at[s].at[i * 4 + c], rsem.at[s])

    def wait_gathers(s, idxv_s):
        for i in range(CHUNK):
            for c in range(4):
                pltpu.make_async_copy(table_ref.at[idxv_s.at[i, c]],
                                      rows_v.at[s].at[i * 4 + c],
                                      rsem.at[s]).wait()

    def compute(s, base):
        hi_mask = jnp.int32(-65536)
        for i in range(CHUNK):
            def head_body(h, carry2, i=i, s=s):
                acc0 = jnp.zeros((16,), jnp.float32)
                acc1 = jnp.zeros((16,), jnp.float32)
                for c in range(4):
                    wv = wgt_v[s, i, c, pl.ds(h * P_TOTAL, P_TOTAL)]
                    for p in range(P_TOTAL):
                        r = h * P_TOTAL + p
                        wsc = wv[p]
                        rv = rows_v[s, i * 4 + c, r, pl.ds(0, 16)]
                        ve = plsc.bitcast(rv << 16, jnp.float32)
                        vo = plsc.bitcast(rv & hi_mask, jnp.float32)
                        acc0 = acc0 + wsc * ve
                        acc1 = acc1 + wsc * vo
                out_v[i, h, pl.ds(0, 16)] = acc0
                out_v[i, h, pl.ds(16, 16)] = acc1
                return carry2
            lax.fori_loop(0, N_HEADS, head_body, 0)
        pltpu.sync_copy(out_v, out_hbm.at[pl.ds(base, CHUNK)])

    def prefetch(s, base, which):
        if which == "idx":
            pltpu.async_copy(idx_hbm.at[pl.ds(base, CHUNK)], idx_v.at[s],
                             isem.at[s])
        else:
            pltpu.async_copy(wgt_hbm.at[pl.ds(base, CHUNK)], wgt_v.at[s],
                             wsem.at[s])

    def wait_prefetch(s):
        pltpu.make_async_copy(idx_hbm.at[pl.ds(0, CHUNK)], idx_v.at[s],
                              isem.at[s]).wait()
        pltpu.make_async_copy(wgt_hbm.at[pl.ds(0, CHUNK)], wgt_v.at[s],
                              wsem.at[s]).wait()

    pltpu.sync_copy(idx_hbm.at[pl.ds(w_base, CHUNK)], idx_v.at[0])
    pltpu.sync_copy(wgt_hbm.at[pl.ds(w_base, CHUNK)], wgt_v.at[0])
    issue_gathers(0, idx_v.at[0])
    prefetch(1, w_base + CHUNK, "idx")
    prefetch(1, w_base + CHUNK, "wgt")

    def chunk_step(c_dyn, s, ns):
        base = w_base + c_dyn * CHUNK
        wait_prefetch(ns)
        issue_gathers(ns, idx_v.at[ns])
        wait_gathers(s, idx_v.at[s])
        prefetch(s, base + 2 * CHUNK, "idx")
        compute(s, base)
        prefetch(s, base + 2 * CHUNK, "wgt")

    def pair_body(k, carry):
        c0 = 2 * k
        chunk_step(c0, 0, 1)
        chunk_step(c0 + 1, 1, 0)
        return carry

    lax.fori_loop(0, (N_CHUNKS - 2) // 2, pair_body, 0)
    wait_prefetch(1)
    issue_gathers(1, idx_v.at[1])
    wait_gathers(0, idx_v.at[0])
    compute(0, w_base + (N_CHUNKS - 2) * CHUNK)
    wait_gathers(1, idx_v.at[1])
    compute(1, w_base + (N_CHUNKS - 1) * CHUNK)


def _tr_kernel(v_ref, o_ref):
    for h in range(N_HEADS):
        o_ref[0, 0, h] = jnp.transpose(v_ref[0, 0, h])


@functools.lru_cache(maxsize=1)
def _make_sc_gather():
    return functools.partial(
        pl.kernel,
        out_type=jax.ShapeDtypeStruct((N_ITEMS, N_HEADS, HEAD_DIM),
                                      jnp.float32),
        mesh=plsc.VectorSubcoreMesh(core_axis_name="c", subcore_axis_name="s",
                                    num_cores=NC, num_subcores=NS),
        scratch_types=[
            pltpu.VMEM((2, CHUNK, 4, 128), jnp.int32),
            pltpu.VMEM((2, CHUNK, 4, 128), jnp.float32),
            pltpu.VMEM((2, CHUNK * 4, 128, HEAD_DIM // 2), jnp.int32),
            pltpu.VMEM((CHUNK, N_HEADS, HEAD_DIM), jnp.float32),
            pltpu.SemaphoreType.DMA((2,)),
            pltpu.SemaphoreType.DMA((2,)),
            pltpu.SemaphoreType.DMA((2,)),
        ],
        compiler_params=pltpu.CompilerParams(use_tc_tiling_on_sc=False,
                                             needs_layout_passes=False),
    )(_sc_gather_kernel)


def kernel(query, reference_points, value, value_spatial_shapes,
           W_off, b_off, W_attn, b_attn):
    del value_spatial_shapes
    Wo = W_off.reshape(D_MODEL, N_HEADS * P_TOTAL, 2)
    Wx, Wy = Wo[:, :, 0], Wo[:, :, 1]
    bo = b_off.reshape(N_HEADS * P_TOTAL, 2)
    bx, by = bo[:, 0].reshape(1, -1), bo[:, 1].reshape(1, -1)
    ba = b_attn.reshape(1, -1)

    idx, wgt = pl.pallas_call(
        _prep_kernel,
        grid=(BS,),
        in_specs=[
            pl.BlockSpec((1, LEN_Q, D_MODEL), lambda b: (b, 0, 0)),
            pl.BlockSpec((1, LEN_Q, 1, 4), lambda b: (b, 0, 0, 0)),
            pl.BlockSpec((D_MODEL, 128), lambda b: (0, 0)),
            pl.BlockSpec((D_MODEL, 128), lambda b: (0, 0)),
            pl.BlockSpec((D_MODEL, 128), lambda b: (0, 0)),
            pl.BlockSpec((1, 128), lambda b: (0, 0)),
            pl.BlockSpec((1, 128), lambda b: (0, 0)),
            pl.BlockSpec((1, 128), lambda b: (0, 0)),
        ],
        out_specs=[
            pl.BlockSpec((1, LEN_Q, 4, 128), lambda b: (b, 0, 0, 0)),
            pl.BlockSpec((1, LEN_Q, 4, 128), lambda b: (b, 0, 0, 0)),
        ],
        out_shape=[
            jax.ShapeDtypeStruct((BS, LEN_Q, 4, 128), jnp.int32),
            jax.ShapeDtypeStruct((BS, LEN_Q, 4, 128), jnp.float32),
        ],
    )(query, reference_points, Wx, Wy, W_attn, bx, by, ba)

    vt = jnp.transpose(value.astype(jnp.bfloat16), (0, 1, 2, 4, 3))
    u16 = jax.lax.bitcast_convert_type(
        vt.reshape(N_LEVELS, BS, N_HEADS, HW, HEAD_DIM // 2, 2), jnp.uint16)
    w32 = (u16[..., 1].astype(jnp.uint32) << 16) | u16[..., 0].astype(jnp.uint32)
    table = jax.lax.bitcast_convert_type(w32, jnp.int32).reshape(
        N_LEVELS * BS * N_HEADS * HW, HEAD_DIM // 2)

    idx = idx.reshape(N_ITEMS, 4, 128)
    wgt = wgt.reshape(N_ITEMS, 4, 128)
    out = _make_sc_gather()(table, idx, wgt)
    out = out.reshape(BS, LEN_Q, N_HEADS, 2, HEAD_DIM // 2)
    return jnp.transpose(out, (0, 1, 2, 4, 3)).reshape(BS, LEN_Q, D_MODEL)


# speedup vs baseline: 3.0292x; 3.0292x over previous
import functools

import jax
import jax.numpy as jnp
from jax import lax
from jax.experimental import pallas as pl
from jax.experimental.pallas import tpu as pltpu
from jax.experimental.pallas import tpu_sc as plsc

BS = 16
LEN_Q = 300
D_MODEL = 256
N_HEADS = 8
HEAD_DIM = 32
N_LEVELS = 4
P_TOTAL = 16
H = 40
W = 40
HW = H * W
N_ITEMS = BS * LEN_Q
ENTRIES = 4 * N_HEADS * P_TOTAL

NC, NS = 2, 16
NW = NC * NS
ITEMS_PER_W = N_ITEMS // NW
CHUNK = 3
N_CHUNKS = ITEMS_PER_W // CHUNK


def _prep_kernel(q_ref, rp_ref, wx_ref, wy_ref, wa_ref, bx_ref, by_ref,
                 ba_ref, idx_ref, wgt_ref):
    b = pl.program_id(0)
    q = q_ref[0]
    sox = jnp.dot(q, wx_ref[...], preferred_element_type=jnp.float32) + bx_ref[0]
    soy = jnp.dot(q, wy_ref[...], preferred_element_type=jnp.float32) + by_ref[0]
    logits = jnp.dot(q, wa_ref[...], preferred_element_type=jnp.float32) + ba_ref[0]

    aw_parts = []
    for h in range(N_HEADS):
        lg = logits[:, h * P_TOTAL:(h + 1) * P_TOTAL]
        e = jnp.exp(lg - jnp.max(lg, axis=1, keepdims=True))
        aw_parts.append(e / jnp.sum(e, axis=1, keepdims=True))
    aw = jnp.concatenate(aw_parts, axis=1)

    rp = rp_ref[0, :, 0, :]
    refx, refy = rp[:, 0:1], rp[:, 1:2]
    refw, refh = rp[:, 2:3], rp[:, 3:4]
    px = refx * W + sox * refw * (0.125 * W) - 0.5
    py = refy * H + soy * refh * (0.125 * H) - 0.5

    x0 = jnp.floor(px)
    y0 = jnp.floor(py)
    fx = px - x0
    fy = py - y0
    x1 = x0 + 1.0
    y1 = y0 + 1.0
    vx0 = ((x0 >= 0.0) & (x0 <= W - 1.0)).astype(jnp.float32)
    vx1 = ((x1 >= 0.0) & (x1 <= W - 1.0)).astype(jnp.float32)
    vy0 = ((y0 >= 0.0) & (y0 <= H - 1.0)).astype(jnp.float32)
    vy1 = ((y1 >= 0.0) & (y1 <= H - 1.0)).astype(jnp.float32)
    x0c = jnp.clip(x0, 0.0, W - 1.0).astype(jnp.int32)
    x1c = jnp.clip(x1, 0.0, W - 1.0).astype(jnp.int32)
    y0c = jnp.clip(y0, 0.0, H - 1.0).astype(jnp.int32)
    y1c = jnp.clip(y1, 0.0, H - 1.0).astype(jnp.int32)

    wx0 = (1.0 - fx) * vx0
    wx1 = fx * vx1
    wy0 = (1.0 - fy) * vy0
    wy1 = fy * vy1

    lane = lax.broadcasted_iota(jnp.int32, (LEN_Q, N_HEADS * P_TOTAL), 1)
    hh = lane >> 4
    lvl = (lane & 15) >> 2
    base = ((lvl * BS + b) * N_HEADS + hh) * HW

    row_y0 = base + y0c * W
    row_y1 = base + y1c * W
    idx_ref[0, :, 0, :] = row_y0 + x0c
    idx_ref[0, :, 1, :] = row_y0 + x1c
    idx_ref[0, :, 2, :] = row_y1 + x0c
    idx_ref[0, :, 3, :] = row_y1 + x1c
    wgt_ref[0, :, 0, :] = aw * wx0 * wy0
    wgt_ref[0, :, 1, :] = aw * wx1 * wy0
    wgt_ref[0, :, 2, :] = aw * wx0 * wy1
    wgt_ref[0, :, 3, :] = aw * wx1 * wy1


def _sc_gather_kernel(table_ref, idx_hbm, wgt_hbm, out_hbm,
                      idx_v, wgt_v, rows_v, out_v, rsem, isem, wsem):
    wid = lax.axis_index("s") * NC + lax.axis_index("c")
    w_base = wid * ITEMS_PER_W

    def issue_gathers(s, idxv_s):
        for i in range(CHUNK):
            for c in range(4):
                pltpu.async_copy(table_ref.at[idxv_s.at[i, c]],
                                 rows_v.at[s].at[i * 4 + c], rsem.at[s])

    def wait_gathers(s, idxv_s):
        for i in range(CHUNK):
            for c in range(4):
                pltpu.make_async_copy(table_ref.at[idxv_s.at[i, c]],
                                      rows_v.at[s].at[i * 4 + c],
                                      rsem.at[s]).wait()

    def compute(s, base):
        for i in range(CHUNK):
            def head_body(h, carry2, i=i, s=s):
                acc0 = jnp.zeros((16,), jnp.float32)
                acc1 = jnp.zeros((16,), jnp.float32)
                for c in range(4):
                    wv = wgt_v[s, i, c, pl.ds(h * P_TOTAL, P_TOTAL)]
                    for p in range(P_TOTAL):
                        r = h * P_TOTAL + p
                        wsc = wv[p]
                        acc0 = acc0 + wsc * rows_v[s, i * 4 + c, r, pl.ds(0, 16)]
                        acc1 = acc1 + wsc * rows_v[s, i * 4 + c, r, pl.ds(16, 16)]
                out_v[i, h, pl.ds(0, 16)] = acc0
                out_v[i, h, pl.ds(16, 16)] = acc1
                return carry2
            lax.fori_loop(0, N_HEADS, head_body, 0, unroll=2)
        pltpu.sync_copy(out_v, out_hbm.at[pl.ds(base, CHUNK)])

    def prefetch(s, base, which):
        if which == "idx":
            pltpu.async_copy(idx_hbm.at[pl.ds(base, CHUNK)], idx_v.at[s],
                             isem.at[s])
        else:
            pltpu.async_copy(wgt_hbm.at[pl.ds(base, CHUNK)], wgt_v.at[s],
                             wsem.at[s])

    def wait_prefetch(s):
        pltpu.make_async_copy(idx_hbm.at[pl.ds(0, CHUNK)], idx_v.at[s],
                              isem.at[s]).wait()
        pltpu.make_async_copy(wgt_hbm.at[pl.ds(0, CHUNK)], wgt_v.at[s],
                              wsem.at[s]).wait()

    pltpu.sync_copy(idx_hbm.at[pl.ds(w_base, CHUNK)], idx_v.at[0])
    pltpu.sync_copy(wgt_hbm.at[pl.ds(w_base, CHUNK)], wgt_v.at[0])
    issue_gathers(0, idx_v.at[0])
    prefetch(1, w_base + CHUNK, "idx")
    prefetch(1, w_base + CHUNK, "wgt")

    def chunk_step(c_dyn, s, ns):
        base = w_base + c_dyn * CHUNK
        wait_prefetch(ns)
        issue_gathers(ns, idx_v.at[ns])
        wait_gathers(s, idx_v.at[s])
        prefetch(s, base + 2 * CHUNK, "idx")
        compute(s, base)
        prefetch(s, base + 2 * CHUNK, "wgt")

    def pair_body(k, carry):
        c0 = 2 * k
        chunk_step(c0, 0, 1)
        chunk_step(c0 + 1, 1, 0)
        return carry

    lax.fori_loop(0, (N_CHUNKS - 2) // 2, pair_body, 0)
    wait_prefetch(1)
    issue_gathers(1, idx_v.at[1])
    wait_gathers(0, idx_v.at[0])
    compute(0, w_base + (N_CHUNKS - 2) * CHUNK)
    wait_gathers(1, idx_v.at[1])
    compute(1, w_base + (N_CHUNKS - 1) * CHUNK)


@functools.lru_cache(maxsize=1)
def _make_sc_gather():
    return functools.partial(
        pl.kernel,
        out_type=jax.ShapeDtypeStruct((N_ITEMS, N_HEADS, HEAD_DIM),
                                      jnp.float32),
        mesh=plsc.VectorSubcoreMesh(core_axis_name="c", subcore_axis_name="s",
                                    num_cores=NC, num_subcores=NS),
        scratch_types=[
            pltpu.VMEM((2, CHUNK, 4, 128), jnp.int32),
            pltpu.VMEM((2, CHUNK, 4, 128), jnp.float32),
            pltpu.VMEM((2, CHUNK * 4, 128, HEAD_DIM), jnp.float32),
            pltpu.VMEM((CHUNK, N_HEADS, HEAD_DIM), jnp.float32),
            pltpu.SemaphoreType.DMA((2,)),
            pltpu.SemaphoreType.DMA((2,)),
            pltpu.SemaphoreType.DMA((2,)),
        ],
        compiler_params=pltpu.CompilerParams(use_tc_tiling_on_sc=False,
                                             needs_layout_passes=False),
    )(_sc_gather_kernel)


def kernel(query, reference_points, value, value_spatial_shapes,
           W_off, b_off, W_attn, b_attn):
    del value_spatial_shapes
    Wo = W_off.reshape(D_MODEL, N_HEADS * P_TOTAL, 2)
    Wx, Wy = Wo[:, :, 0], Wo[:, :, 1]
    bo = b_off.reshape(N_HEADS * P_TOTAL, 2)
    bx, by = bo[:, 0].reshape(1, -1), bo[:, 1].reshape(1, -1)
    ba = b_attn.reshape(1, -1)

    idx, wgt = pl.pallas_call(
        _prep_kernel,
        grid=(BS,),
        in_specs=[
            pl.BlockSpec((1, LEN_Q, D_MODEL), lambda b: (b, 0, 0)),
            pl.BlockSpec((1, LEN_Q, 1, 4), lambda b: (b, 0, 0, 0)),
            pl.BlockSpec((D_MODEL, 128), lambda b: (0, 0)),
            pl.BlockSpec((D_MODEL, 128), lambda b: (0, 0)),
            pl.BlockSpec((D_MODEL, 128), lambda b: (0, 0)),
            pl.BlockSpec((1, 128), lambda b: (0, 0)),
            pl.BlockSpec((1, 128), lambda b: (0, 0)),
            pl.BlockSpec((1, 128), lambda b: (0, 0)),
        ],
        out_specs=[
            pl.BlockSpec((1, LEN_Q, 4, 128), lambda b: (b, 0, 0, 0)),
            pl.BlockSpec((1, LEN_Q, 4, 128), lambda b: (b, 0, 0, 0)),
        ],
        out_shape=[
            jax.ShapeDtypeStruct((BS, LEN_Q, 4, 128), jnp.int32),
            jax.ShapeDtypeStruct((BS, LEN_Q, 4, 128), jnp.float32),
        ],
    )(query, reference_points, Wx, Wy, W_attn, bx, by, ba)

    table = jnp.transpose(value, (0, 1, 2, 4, 3)).reshape(
        N_LEVELS * BS * N_HEADS * HW, HEAD_DIM)

    idx = idx.reshape(N_ITEMS, 4, 128)
    wgt = wgt.reshape(N_ITEMS, 4, 128)
    out = _make_sc_gather()(table, idx, wgt)
    return out.reshape(BS, LEN_Q, D_MODEL)


# speedup vs baseline: 3.0633x; 1.0112x over previous
import functools

import jax
import jax.numpy as jnp
from jax import lax
from jax.experimental import pallas as pl
from jax.experimental.pallas import tpu as pltpu
from jax.experimental.pallas import tpu_sc as plsc

BS = 16
LEN_Q = 300
D_MODEL = 256
N_HEADS = 8
HEAD_DIM = 32
N_LEVELS = 4
P_TOTAL = 16
H = 40
W = 40
HW = H * W
N_ITEMS = BS * LEN_Q
ENTRIES = 4 * N_HEADS * P_TOTAL

NC, NS = 2, 16
NW = NC * NS
ITEMS_PER_W = N_ITEMS // NW
CHUNK = 3
N_CHUNKS = ITEMS_PER_W // CHUNK


def _prep_kernel(q_ref, rp_ref, wx_ref, wy_ref, wa_ref, bx_ref, by_ref,
                 ba_ref, idx_ref, wgt_ref):
    b = pl.program_id(0)
    q = q_ref[0]
    sox = jnp.dot(q, wx_ref[...], preferred_element_type=jnp.float32) + bx_ref[0]
    soy = jnp.dot(q, wy_ref[...], preferred_element_type=jnp.float32) + by_ref[0]
    logits = jnp.dot(q, wa_ref[...], preferred_element_type=jnp.float32) + ba_ref[0]

    aw_parts = []
    for h in range(N_HEADS):
        lg = logits[:, h * P_TOTAL:(h + 1) * P_TOTAL]
        e = jnp.exp(lg - jnp.max(lg, axis=1, keepdims=True))
        aw_parts.append(e / jnp.sum(e, axis=1, keepdims=True))
    aw = jnp.concatenate(aw_parts, axis=1)

    rp = rp_ref[0, :, 0, :]
    refx, refy = rp[:, 0:1], rp[:, 1:2]
    refw, refh = rp[:, 2:3], rp[:, 3:4]
    px = refx * W + sox * refw * (0.125 * W) - 0.5
    py = refy * H + soy * refh * (0.125 * H) - 0.5

    x0 = jnp.floor(px)
    y0 = jnp.floor(py)
    fx = px - x0
    fy = py - y0
    x1 = x0 + 1.0
    y1 = y0 + 1.0
    vx0 = ((x0 >= 0.0) & (x0 <= W - 1.0)).astype(jnp.float32)
    vx1 = ((x1 >= 0.0) & (x1 <= W - 1.0)).astype(jnp.float32)
    vy0 = ((y0 >= 0.0) & (y0 <= H - 1.0)).astype(jnp.float32)
    vy1 = ((y1 >= 0.0) & (y1 <= H - 1.0)).astype(jnp.float32)
    x0c = jnp.clip(x0, 0.0, W - 1.0).astype(jnp.int32)
    x1c = jnp.clip(x1, 0.0, W - 1.0).astype(jnp.int32)
    y0c = jnp.clip(y0, 0.0, H - 1.0).astype(jnp.int32)
    y1c = jnp.clip(y1, 0.0, H - 1.0).astype(jnp.int32)

    wx0 = (1.0 - fx) * vx0
    wx1 = fx * vx1
    wy0 = (1.0 - fy) * vy0
    wy1 = fy * vy1

    lane = lax.broadcasted_iota(jnp.int32, (LEN_Q, N_HEADS * P_TOTAL), 1)
    hh = lane >> 4
    lvl = (lane & 15) >> 2
    base = ((lvl * BS + b) * N_HEADS + hh) * HW

    row_y0 = base + y0c * W
    row_y1 = base + y1c * W
    idx_ref[0, :, 0, :] = row_y0 + x0c
    idx_ref[0, :, 1, :] = row_y0 + x1c
    idx_ref[0, :, 2, :] = row_y1 + x0c
    idx_ref[0, :, 3, :] = row_y1 + x1c
    wgt_ref[0, :, 0, :] = aw * wx0 * wy0
    wgt_ref[0, :, 1, :] = aw * wx1 * wy0
    wgt_ref[0, :, 2, :] = aw * wx0 * wy1
    wgt_ref[0, :, 3, :] = aw * wx1 * wy1


def _sc_gather_kernel(table_ref, idx_hbm, wgt_hbm, out_hbm,
                      idx_v, wgt_v, rows_v, out_v, rsem, isem, wsem):
    wid = lax.axis_index("s") * NC + lax.axis_index("c")
    w_base = wid * ITEMS_PER_W

    def issue_gathers(s, idxv_s):
        for i in range(CHUNK):
            for c in range(4):
                pltpu.async_copy(table_ref.at[idxv_s.at[i, c]],
                                 rows_v.at[s].at[i * 4 + c], rsem.at[s])

    def wait_gathers(s, idxv_s):
        for i in range(CHUNK):
            for c in range(4):
                pltpu.make_async_copy(table_ref.at[idxv_s.at[i, c]],
                                      rows_v.at[s].at[i * 4 + c],
                                      rsem.at[s]).wait()

    def compute(s, base):
        for i in range(CHUNK):
            def head_body(h, carry2, i=i, s=s):
                acc0 = jnp.zeros((16,), jnp.float32)
                acc1 = jnp.zeros((16,), jnp.float32)
                for c in range(4):
                    wv = wgt_v[s, i, c, pl.ds(h * P_TOTAL, P_TOTAL)]
                    for p in range(P_TOTAL):
                        r = h * P_TOTAL + p
                        wsc = wv[p]
                        acc0 = acc0 + wsc * rows_v[s, i * 4 + c, r, pl.ds(0, 16)]
                        acc1 = acc1 + wsc * rows_v[s, i * 4 + c, r, pl.ds(16, 16)]
                out_v[i, h, pl.ds(0, 16)] = acc0
                out_v[i, h, pl.ds(16, 16)] = acc1
                return carry2
            lax.fori_loop(0, N_HEADS, head_body, 0)
        pltpu.sync_copy(out_v, out_hbm.at[pl.ds(base, CHUNK)])

    def prefetch(s, base, which):
        if which == "idx":
            pltpu.async_copy(idx_hbm.at[pl.ds(base, CHUNK)], idx_v.at[s],
                             isem.at[s])
        else:
            pltpu.async_copy(wgt_hbm.at[pl.ds(base, CHUNK)], wgt_v.at[s],
                             wsem.at[s])

    def wait_prefetch(s):
        pltpu.make_async_copy(idx_hbm.at[pl.ds(0, CHUNK)], idx_v.at[s],
                              isem.at[s]).wait()
        pltpu.make_async_copy(wgt_hbm.at[pl.ds(0, CHUNK)], wgt_v.at[s],
                              wsem.at[s]).wait()

    pltpu.sync_copy(idx_hbm.at[pl.ds(w_base, CHUNK)], idx_v.at[0])
    pltpu.sync_copy(wgt_hbm.at[pl.ds(w_base, CHUNK)], wgt_v.at[0])
    issue_gathers(0, idx_v.at[0])
    prefetch(1, w_base + CHUNK, "idx")
    prefetch(1, w_base + CHUNK, "wgt")

    def chunk_step(c_dyn, s, ns):
        base = w_base + c_dyn * CHUNK
        wait_prefetch(ns)
        issue_gathers(ns, idx_v.at[ns])
        wait_gathers(s, idx_v.at[s])
        prefetch(s, base + 2 * CHUNK, "idx")
        compute(s, base)
        prefetch(s, base + 2 * CHUNK, "wgt")

    def pair_body(k, carry):
        c0 = 2 * k
        chunk_step(c0, 0, 1)
        chunk_step(c0 + 1, 1, 0)
        return carry

    lax.fori_loop(0, (N_CHUNKS - 2) // 2, pair_body, 0)
    wait_prefetch(1)
    issue_gathers(1, idx_v.at[1])
    wait_gathers(0, idx_v.at[0])
    compute(0, w_base + (N_CHUNKS - 2) * CHUNK)
    wait_gathers(1, idx_v.at[1])
    compute(1, w_base + (N_CHUNKS - 1) * CHUNK)


@functools.lru_cache(maxsize=1)
def _make_sc_gather():
    return functools.partial(
        pl.kernel,
        out_type=jax.ShapeDtypeStruct((N_ITEMS, N_HEADS, HEAD_DIM),
                                      jnp.float32),
        mesh=plsc.VectorSubcoreMesh(core_axis_name="c", subcore_axis_name="s",
                                    num_cores=NC, num_subcores=NS),
        scratch_types=[
            pltpu.VMEM((2, CHUNK, 4, 128), jnp.int32),
            pltpu.VMEM((2, CHUNK, 4, 128), jnp.float32),
            pltpu.VMEM((2, CHUNK * 4, 128, HEAD_DIM), jnp.float32),
            pltpu.VMEM((CHUNK, N_HEADS, HEAD_DIM), jnp.float32),
            pltpu.SemaphoreType.DMA((2,)),
            pltpu.SemaphoreType.DMA((2,)),
            pltpu.SemaphoreType.DMA((2,)),
        ],
        compiler_params=pltpu.CompilerParams(use_tc_tiling_on_sc=False,
                                             needs_layout_passes=False),
    )(_sc_gather_kernel)


def kernel(query, reference_points, value, value_spatial_shapes,
           W_off, b_off, W_attn, b_attn):
    del value_spatial_shapes
    Wo = W_off.reshape(D_MODEL, N_HEADS * P_TOTAL, 2)
    Wx, Wy = Wo[:, :, 0], Wo[:, :, 1]
    bo = b_off.reshape(N_HEADS * P_TOTAL, 2)
    bx, by = bo[:, 0].reshape(1, -1), bo[:, 1].reshape(1, -1)
    ba = b_attn.reshape(1, -1)

    idx, wgt = pl.pallas_call(
        _prep_kernel,
        grid=(BS,),
        in_specs=[
            pl.BlockSpec((1, LEN_Q, D_MODEL), lambda b: (b, 0, 0)),
            pl.BlockSpec((1, LEN_Q, 1, 4), lambda b: (b, 0, 0, 0)),
            pl.BlockSpec((D_MODEL, 128), lambda b: (0, 0)),
            pl.BlockSpec((D_MODEL, 128), lambda b: (0, 0)),
            pl.BlockSpec((D_MODEL, 128), lambda b: (0, 0)),
            pl.BlockSpec((1, 128), lambda b: (0, 0)),
            pl.BlockSpec((1, 128), lambda b: (0, 0)),
            pl.BlockSpec((1, 128), lambda b: (0, 0)),
        ],
        out_specs=[
            pl.BlockSpec((1, LEN_Q, 4, 128), lambda b: (b, 0, 0, 0)),
            pl.BlockSpec((1, LEN_Q, 4, 128), lambda b: (b, 0, 0, 0)),
        ],
        out_shape=[
            jax.ShapeDtypeStruct((BS, LEN_Q, 4, 128), jnp.int32),
            jax.ShapeDtypeStruct((BS, LEN_Q, 4, 128), jnp.float32),
        ],
    )(query, reference_points, Wx, Wy, W_attn, bx, by, ba)

    table = jnp.transpose(value, (0, 1, 2, 4, 3)).reshape(
        N_LEVELS * BS * N_HEADS * HW, HEAD_DIM)

    idx = idx.reshape(N_ITEMS, 4, 128)
    wgt = wgt.reshape(N_ITEMS, 4, 128)
    out = _make_sc_gather()(table, idx, wgt)
    return out.reshape(BS, LEN_Q, D_MODEL)


# speedup vs baseline: 3.0656x; 1.0008x over previous
import functools

import jax
import jax.numpy as jnp
from jax import lax
from jax.experimental import pallas as pl
from jax.experimental.pallas import tpu as pltpu
from jax.experimental.pallas import tpu_sc as plsc

BS = 16
LEN_Q = 300
D_MODEL = 256
N_HEADS = 8
HEAD_DIM = 32
N_LEVELS = 4
P_TOTAL = 16
H = 40
W = 40
HW = H * W
N_ITEMS = BS * LEN_Q
ENTRIES = 4 * N_HEADS * P_TOTAL

NC, NS = 2, 16
NW = NC * NS
ITEMS_PER_W = N_ITEMS // NW
CHUNK = 3
N_CHUNKS = ITEMS_PER_W // CHUNK


def _prep_kernel(q_ref, rp_ref, wx_ref, wy_ref, wa_ref, bx_ref, by_ref,
                 ba_ref, idx_ref, wgt_ref):
    b = pl.program_id(0)
    q = q_ref[0]
    sox = jnp.dot(q, wx_ref[...], preferred_element_type=jnp.float32) + bx_ref[0]
    soy = jnp.dot(q, wy_ref[...], preferred_element_type=jnp.float32) + by_ref[0]
    logits = jnp.dot(q, wa_ref[...], preferred_element_type=jnp.float32) + ba_ref[0]

    rr = lax.broadcasted_iota(jnp.int32, (128, 128), 0) >> 4
    cc = lax.broadcasted_iota(jnp.int32, (128, 128), 1) >> 4
    bd = (rr == cc).astype(jnp.float32)
    e = jnp.exp(logits - jnp.max(logits, axis=1, keepdims=True))
    aw = e / jnp.dot(e, bd, preferred_element_type=jnp.float32)

    rp = rp_ref[0, :, 0, :]
    refx, refy = rp[:, 0:1], rp[:, 1:2]
    refw, refh = rp[:, 2:3], rp[:, 3:4]
    px = refx * W + sox * refw * (0.125 * W) - 0.5
    py = refy * H + soy * refh * (0.125 * H) - 0.5

    x0 = jnp.floor(px)
    y0 = jnp.floor(py)
    fx = px - x0
    fy = py - y0
    x1 = x0 + 1.0
    y1 = y0 + 1.0
    vx0 = ((x0 >= 0.0) & (x0 <= W - 1.0)).astype(jnp.float32)
    vx1 = ((x1 >= 0.0) & (x1 <= W - 1.0)).astype(jnp.float32)
    vy0 = ((y0 >= 0.0) & (y0 <= H - 1.0)).astype(jnp.float32)
    vy1 = ((y1 >= 0.0) & (y1 <= H - 1.0)).astype(jnp.float32)
    x0c = jnp.clip(x0, 0.0, W - 1.0).astype(jnp.int32)
    x1c = jnp.clip(x1, 0.0, W - 1.0).astype(jnp.int32)
    y0c = jnp.clip(y0, 0.0, H - 1.0).astype(jnp.int32)
    y1c = jnp.clip(y1, 0.0, H - 1.0).astype(jnp.int32)

    wx0 = (1.0 - fx) * vx0
    wx1 = fx * vx1
    wy0 = (1.0 - fy) * vy0
    wy1 = fy * vy1

    lane = lax.broadcasted_iota(jnp.int32, (LEN_Q, N_HEADS * P_TOTAL), 1)
    hh = lane >> 4
    lvl = (lane & 15) >> 2
    base = ((lvl * BS + b) * N_HEADS + hh) * HW

    row_y0 = base + y0c * W
    row_y1 = base + y1c * W
    idx_ref[0, :, 0, :] = row_y0 + x0c
    idx_ref[0, :, 1, :] = row_y0 + x1c
    idx_ref[0, :, 2, :] = row_y1 + x0c
    idx_ref[0, :, 3, :] = row_y1 + x1c
    wgt_ref[0, :, 0, :] = aw * wx0 * wy0
    wgt_ref[0, :, 1, :] = aw * wx1 * wy0
    wgt_ref[0, :, 2, :] = aw * wx0 * wy1
    wgt_ref[0, :, 3, :] = aw * wx1 * wy1


def _sc_gather_kernel(table_ref, idx_hbm, wgt_hbm, out_hbm,
                      idx_v, wgt_v, rows_v, out_v, rsem, isem, wsem):
    wid = lax.axis_index("s") * NC + lax.axis_index("c")
    w_base = wid * ITEMS_PER_W

    def issue_gathers(s, idxv_s):
        for i in range(CHUNK):
            for c in range(4):
                pltpu.async_copy(table_ref.at[idxv_s.at[i, c]],
                                 rows_v.at[s].at[i * 4 + c], rsem.at[s])

    def wait_gathers(s, idxv_s):
        for i in range(CHUNK):
            for c in range(4):
                pltpu.make_async_copy(table_ref.at[idxv_s.at[i, c]],
                                      rows_v.at[s].at[i * 4 + c],
                                      rsem.at[s]).wait()

    def compute(s, base):
        for i in range(CHUNK):
            def head_body(h, carry2, i=i, s=s):
                acc0 = jnp.zeros((16,), jnp.float32)
                acc1 = jnp.zeros((16,), jnp.float32)
                for c in range(4):
                    wv = wgt_v[s, i, c, pl.ds(h * P_TOTAL, P_TOTAL)]
                    for p in range(P_TOTAL):
                        r = h * P_TOTAL + p
                        wsc = wv[p]
                        acc0 = acc0 + wsc * rows_v[s, i * 4 + c, r, pl.ds(0, 16)]
                        acc1 = acc1 + wsc * rows_v[s, i * 4 + c, r, pl.ds(16, 16)]
                out_v[i, h, pl.ds(0, 16)] = acc0
                out_v[i, h, pl.ds(16, 16)] = acc1
                return carry2
            lax.fori_loop(0, N_HEADS, head_body, 0)
        pltpu.sync_copy(out_v, out_hbm.at[pl.ds(base, CHUNK)])

    def prefetch(s, base, which):
        if which == "idx":
            pltpu.async_copy(idx_hbm.at[pl.ds(base, CHUNK)], idx_v.at[s],
                             isem.at[s])
        else:
            pltpu.async_copy(wgt_hbm.at[pl.ds(base, CHUNK)], wgt_v.at[s],
                             wsem.at[s])

    def wait_prefetch(s):
        pltpu.make_async_copy(idx_hbm.at[pl.ds(0, CHUNK)], idx_v.at[s],
                              isem.at[s]).wait()
        pltpu.make_async_copy(wgt_hbm.at[pl.ds(0, CHUNK)], wgt_v.at[s],
                              wsem.at[s]).wait()

    pltpu.sync_copy(idx_hbm.at[pl.ds(w_base, CHUNK)], idx_v.at[0])
    pltpu.sync_copy(wgt_hbm.at[pl.ds(w_base, CHUNK)], wgt_v.at[0])
    issue_gathers(0, idx_v.at[0])
    prefetch(1, w_base + CHUNK, "idx")
    prefetch(1, w_base + CHUNK, "wgt")

    def chunk_step(c_dyn, s, ns):
        base = w_base + c_dyn * CHUNK
        wait_prefetch(ns)
        issue_gathers(ns, idx_v.at[ns])
        wait_gathers(s, idx_v.at[s])
        prefetch(s, base + 2 * CHUNK, "idx")
        compute(s, base)
        prefetch(s, base + 2 * CHUNK, "wgt")

    def pair_body(k, carry):
        c0 = 2 * k
        chunk_step(c0, 0, 1)
        chunk_step(c0 + 1, 1, 0)
        return carry

    lax.fori_loop(0, (N_CHUNKS - 2) // 2, pair_body, 0)
    wait_prefetch(1)
    issue_gathers(1, idx_v.at[1])
    wait_gathers(0, idx_v.at[0])
    compute(0, w_base + (N_CHUNKS - 2) * CHUNK)
    wait_gathers(1, idx_v.at[1])
    compute(1, w_base + (N_CHUNKS - 1) * CHUNK)


@functools.lru_cache(maxsize=1)
def _make_sc_gather():
    return functools.partial(
        pl.kernel,
        out_type=jax.ShapeDtypeStruct((N_ITEMS, N_HEADS, HEAD_DIM),
                                      jnp.float32),
        mesh=plsc.VectorSubcoreMesh(core_axis_name="c", subcore_axis_name="s",
                                    num_cores=NC, num_subcores=NS),
        scratch_types=[
            pltpu.VMEM((2, CHUNK, 4, 128), jnp.int32),
            pltpu.VMEM((2, CHUNK, 4, 128), jnp.float32),
            pltpu.VMEM((2, CHUNK * 4, 128, HEAD_DIM), jnp.float32),
            pltpu.VMEM((CHUNK, N_HEADS, HEAD_DIM), jnp.float32),
            pltpu.SemaphoreType.DMA((2,)),
            pltpu.SemaphoreType.DMA((2,)),
            pltpu.SemaphoreType.DMA((2,)),
        ],
        compiler_params=pltpu.CompilerParams(use_tc_tiling_on_sc=False,
                                             needs_layout_passes=False),
    )(_sc_gather_kernel)


def kernel(query, reference_points, value, value_spatial_shapes,
           W_off, b_off, W_attn, b_attn):
    del value_spatial_shapes
    Wo = W_off.reshape(D_MODEL, N_HEADS * P_TOTAL, 2)
    Wx, Wy = Wo[:, :, 0], Wo[:, :, 1]
    bo = b_off.reshape(N_HEADS * P_TOTAL, 2)
    bx, by = bo[:, 0].reshape(1, -1), bo[:, 1].reshape(1, -1)
    ba = b_attn.reshape(1, -1)

    idx, wgt = pl.pallas_call(
        _prep_kernel,
        grid=(BS,),
        in_specs=[
            pl.BlockSpec((1, LEN_Q, D_MODEL), lambda b: (b, 0, 0)),
            pl.BlockSpec((1, LEN_Q, 1, 4), lambda b: (b, 0, 0, 0)),
            pl.BlockSpec((D_MODEL, 128), lambda b: (0, 0)),
            pl.BlockSpec((D_MODEL, 128), lambda b: (0, 0)),
            pl.BlockSpec((D_MODEL, 128), lambda b: (0, 0)),
            pl.BlockSpec((1, 128), lambda b: (0, 0)),
            pl.BlockSpec((1, 128), lambda b: (0, 0)),
            pl.BlockSpec((1, 128), lambda b: (0, 0)),
        ],
        out_specs=[
            pl.BlockSpec((1, LEN_Q, 4, 128), lambda b: (b, 0, 0, 0)),
            pl.BlockSpec((1, LEN_Q, 4, 128), lambda b: (b, 0, 0, 0)),
        ],
        out_shape=[
            jax.ShapeDtypeStruct((BS, LEN_Q, 4, 128), jnp.int32),
            jax.ShapeDtypeStruct((BS, LEN_Q, 4, 128), jnp.float32),
        ],
    )(query, reference_points, Wx, Wy, W_attn, bx, by, ba)

    table = jnp.transpose(value, (0, 1, 2, 4, 3)).reshape(
        N_LEVELS * BS * N_HEADS * HW, HEAD_DIM)

    idx = idx.reshape(N_ITEMS, 4, 128)
    wgt = wgt.reshape(N_ITEMS, 4, 128)
    out = _make_sc_gather()(table, idx, wgt)
    return out.reshape(BS, LEN_Q, D_MODEL)
